# Initial kernel scaffold; baseline (speedup 1.0000x reference)
import jax
import jax.numpy as jnp
from jax.experimental import pallas as pl


def kernel(x, edge_index, batch, W1, b1, W2, b2, W3, b3):
    raise NotImplementedError("write your pallas kernel here")


# speedup vs baseline: 15.4058x; 15.4058x over previous
import functools

import jax
import jax.numpy as jnp
from jax import lax
from jax.experimental import pallas as pl
from jax.experimental.pallas import tpu as pltpu
from jax.experimental.pallas import tpu_sc as plsc

N = 10000
D = 128
G = 64
E = 320000

NC = 2
NS = 16
NW = NC * NS

NPAD = 10240
NB = NPAD // 128
CHUNK = 128
CPW = 80
EPAD = NW * CPW * CHUNK
RPS = NPAD // NS


def _sc_degree_body(col_hbm, out_hbm, idx_v, ones_v, z_v, deg_sh):
    c = lax.axis_index("c")
    s = lax.axis_index("s")
    wid = c * NS + s
    one16 = jnp.ones((16,), jnp.float32)
    zero16 = jnp.zeros((16,), jnp.float32)
    for i in range(CHUNK // 16):
        ones_v[pl.ds(i * 16, 16)] = one16
    for i in range(RPS // 16):
        z_v[pl.ds(i * 16, 16)] = zero16
    pltpu.sync_copy(z_v, deg_sh.at[pl.ds(s * RPS, RPS)])
    pltpu.sync_copy(col_hbm.at[wid], idx_v)
    plsc.subcore_barrier()

    def step(j, carry):
        pltpu.sync_copy(ones_v, deg_sh.at[idx_v.at[j]], add=True)
        return carry

    lax.fori_loop(0, CPW, step, 0)
    plsc.subcore_barrier()
    pltpu.sync_copy(deg_sh.at[pl.ds(s * RPS, RPS)],
                    out_hbm.at[c, pl.ds(s * RPS, RPS)])


def _sc_scatter_body(row_hbm, col_hbm, hp_hbm, zero_hbm, out_hbm,
                     ridx_v, cidx_v, buf_v, s_sh):
    c = lax.axis_index("c")
    s = lax.axis_index("s")
    wid = c * NS + s
    pltpu.sync_copy(zero_hbm.at[pl.ds(s * RPS, RPS)],
                    s_sh.at[pl.ds(s * RPS, RPS)])
    pltpu.sync_copy(row_hbm.at[wid], ridx_v)
    pltpu.sync_copy(col_hbm.at[wid], cidx_v)
    plsc.subcore_barrier()

    def step(j, carry):
        pltpu.sync_copy(hp_hbm.at[ridx_v.at[j]], buf_v)
        pltpu.sync_copy(buf_v, s_sh.at[cidx_v.at[j]], add=True)
        return carry

    lax.fori_loop(0, CPW, step, 0)
    plsc.subcore_barrier()
    pltpu.sync_copy(s_sh.at[pl.ds(s * RPS, RPS)],
                    out_hbm.at[c, pl.ds(s * RPS, RPS)])


@functools.cache
def _sc_kernels():
    mesh = plsc.VectorSubcoreMesh(core_axis_name="c", subcore_axis_name="s",
                                  num_cores=NC, num_subcores=NS)
    deg = pl.kernel(
        _sc_degree_body,
        out_type=jax.ShapeDtypeStruct((NC, NPAD), jnp.float32),
        mesh=mesh,
        scratch_types=[
            pltpu.VMEM((CPW, CHUNK), jnp.int32),
            pltpu.VMEM((CHUNK,), jnp.float32),
            pltpu.VMEM((RPS,), jnp.float32),
            pltpu.VMEM_SHARED((NPAD,), jnp.float32),
        ],
    )
    scatter = pl.kernel(
        _sc_scatter_body,
        out_type=jax.ShapeDtypeStruct((NC, NPAD, D), jnp.float32),
        mesh=mesh,
        scratch_types=[
            pltpu.VMEM((CPW, CHUNK), jnp.int32),
            pltpu.VMEM((CPW, CHUNK), jnp.int32),
            pltpu.VMEM((CHUNK, D), jnp.float32),
            pltpu.VMEM_SHARED((NPAD, D), jnp.float32),
        ],
    )
    return deg, scatter


def _dis(degp_ref):
    return lax.rsqrt(degp_ref[0] + degp_ref[1] + 1.0)


def _mm1_body(x_ref, w_ref, degp_ref, out_ref):
    h = jnp.dot(x_ref[...], w_ref[...], preferred_element_type=jnp.float32)
    out_ref[...] = h * _dis(degp_ref)


def _pool_block(batch_ref, xl):
    bb = batch_ref[0]
    gids = lax.broadcasted_iota(jnp.int32, (G, CHUNK), 0)
    mask = (gids == bb).astype(jnp.float32)
    return jnp.dot(mask, xl, preferred_element_type=jnp.float32)


def _layer_body(s_ref, hp_ref, degp_ref, b_ref, w_ref, batch_ref,
                hpn_ref, p_ref):
    j = pl.program_id(0)
    dis = _dis(degp_ref)
    xl = jnp.maximum((s_ref[0] + s_ref[1] + hp_ref[...]) * dis + b_ref[...],
                     0.0)
    hpn_ref[...] = jnp.dot(xl, w_ref[...],
                           preferred_element_type=jnp.float32) * dis

    @pl.when(j == 0)
    def _():
        p_ref[...] = jnp.zeros_like(p_ref)

    p_ref[...] += _pool_block(batch_ref, xl)


def _final_body(s_ref, hp_ref, degp_ref, b_ref, batch_ref, p_ref):
    j = pl.program_id(0)
    dis = _dis(degp_ref)
    xl = jnp.maximum((s_ref[0] + s_ref[1] + hp_ref[...]) * dis + b_ref[...],
                     0.0)

    @pl.when(j == 0)
    def _():
        p_ref[...] = jnp.zeros_like(p_ref)

    p_ref[...] += _pool_block(batch_ref, xl)


_x_spec = pl.BlockSpec((CHUNK, D), lambda j: (j, 0))
_w_spec = pl.BlockSpec((D, D), lambda j: (0, 0))
_deg_spec = pl.BlockSpec((NC, CHUNK, 1), lambda j: (0, j, 0))
_s_spec = pl.BlockSpec((NC, CHUNK, D), lambda j: (0, j, 0))
_b_spec = pl.BlockSpec((1, D), lambda j: (0, 0))
_batch_spec = pl.BlockSpec((1, 1, CHUNK), lambda j: (j, 0, 0))
_p_spec = pl.BlockSpec((G, D), lambda j: (0, 0))

_mm1 = pl.pallas_call(
    _mm1_body,
    grid=(NB,),
    in_specs=[_x_spec, _w_spec, _deg_spec],
    out_specs=_x_spec,
    out_shape=jax.ShapeDtypeStruct((NPAD, D), jnp.float32),
)

_layer = pl.pallas_call(
    _layer_body,
    grid=(NB,),
    in_specs=[_s_spec, _x_spec, _deg_spec, _b_spec, _w_spec, _batch_spec],
    out_specs=[_x_spec, _p_spec],
    out_shape=[jax.ShapeDtypeStruct((NPAD, D), jnp.float32),
               jax.ShapeDtypeStruct((G, D), jnp.float32)],
)

_final = pl.pallas_call(
    _final_body,
    grid=(NB,),
    in_specs=[_s_spec, _x_spec, _deg_spec, _b_spec, _batch_spec],
    out_specs=_p_spec,
    out_shape=jax.ShapeDtypeStruct((G, D), jnp.float32),
)


def kernel(x, edge_index, batch, W1, b1, W2, b2, W3, b3):
    row = edge_index[0].astype(jnp.int32)
    col = edge_index[1].astype(jnp.int32)
    pad = jnp.arange(EPAD - E, dtype=jnp.int32) % (NPAD - N) + N
    rowp = jnp.concatenate([row, pad]).reshape(NW, CPW, CHUNK)
    colp = jnp.concatenate([col, pad]).reshape(NW, CPW, CHUNK)
    x_pad = jnp.pad(x, ((0, NPAD - N), (0, 0)))
    batch_pad = jnp.concatenate(
        [batch.astype(jnp.int32), jnp.full((NPAD - N,), -1, jnp.int32)]
    ).reshape(NB, 1, CHUNK)
    zeros = jnp.zeros((NPAD, D), jnp.float32)

    _sc_degree, _sc_scatter = _sc_kernels()
    degp = _sc_degree(colp).reshape(NC, NPAD, 1)
    hp1 = _mm1(x_pad, W1, degp)
    S1 = _sc_scatter(rowp, colp, hp1, zeros)
    hp2, p1 = _layer(S1, hp1, degp, b1.reshape(1, D), W2, batch_pad)
    S2 = _sc_scatter(rowp, colp, hp2, zeros)
    hp3, p2 = _layer(S2, hp2, degp, b2.reshape(1, D), W3, batch_pad)
    S3 = _sc_scatter(rowp, colp, hp3, zeros)
    p3 = _final(S3, hp3, degp, b3.reshape(1, D), batch_pad)
    return jnp.concatenate([p1, p2, p3], axis=1)


# speedup vs baseline: 19.6969x; 1.2785x over previous
import functools

import jax
import jax.numpy as jnp
from jax import lax
from jax.experimental import pallas as pl
from jax.experimental.pallas import tpu as pltpu
from jax.experimental.pallas import tpu_sc as plsc

N = 10000
D = 128
G = 64
E = 320000

NC = 2
NS = 16
NW = NC * NS

NPAD = 10240
NB = NPAD // 128
CHUNK = 128
CPW = 80
EPAD = NW * CPW * CHUNK
KI = 16
RPS = NPAD // NS


def _sc_degree_body(col_hbm, out_hbm, idx_v, ones_v, z_v, deg_sh):
    c = lax.axis_index("c")
    s = lax.axis_index("s")
    wid = c * NS + s
    one16 = jnp.ones((16,), jnp.float32)
    zero16 = jnp.zeros((16,), jnp.float32)
    for i in range(CHUNK // 16):
        ones_v[pl.ds(i * 16, 16)] = one16
    for i in range(RPS // 16):
        z_v[pl.ds(i * 16, 16)] = zero16
    pltpu.sync_copy(z_v, deg_sh.at[pl.ds(s * RPS, RPS)])
    pltpu.sync_copy(col_hbm.at[wid], idx_v)
    plsc.subcore_barrier()

    def step(j, carry):
        pltpu.sync_copy(ones_v, deg_sh.at[idx_v.at[j]], add=True)
        return carry

    lax.fori_loop(0, CPW, step, 0)
    plsc.subcore_barrier()
    pltpu.sync_copy(deg_sh.at[pl.ds(s * RPS, RPS)],
                    out_hbm.at[c, pl.ds(s * RPS, RPS)])


def _sc_scatter_body(row_hbm, col_hbm, hp_hbm, zero_hbm, out_hbm,
                     ridx_v, cidx_v, buf_v, s_sh, sems):
    c = lax.axis_index("c")
    s = lax.axis_index("s")
    wid = c * NS + s
    pltpu.sync_copy(zero_hbm.at[pl.ds(s * RPS, RPS)],
                    s_sh.at[pl.ds(s * RPS, RPS)])
    plsc.subcore_barrier()

    def _wait(b):
        pltpu.make_async_copy(hp_hbm.at[pl.ds(0, CHUNK)], buf_v.at[b],
                              sems.at[b]).wait()

    def group(g, carry):
        goff = pl.multiple_of(g * KI, 8)
        pltpu.sync_copy(row_hbm.at[wid, pl.ds(goff, KI)], ridx_v)
        pltpu.sync_copy(col_hbm.at[wid, pl.ds(goff, KI)], cidx_v)
        pltpu.async_copy(hp_hbm.at[ridx_v.at[0]], buf_v.at[0], sems.at[0])

        def step(h, carry2):
            j0 = 2 * h
            pltpu.async_copy(hp_hbm.at[ridx_v.at[j0 + 1]], buf_v.at[1],
                             sems.at[1])
            _wait(0)
            pltpu.sync_copy(buf_v.at[0], s_sh.at[cidx_v.at[j0]], add=True)

            @pl.when(h < KI // 2 - 1)
            def _():
                pltpu.async_copy(hp_hbm.at[ridx_v.at[j0 + 2]], buf_v.at[0],
                                 sems.at[0])

            _wait(1)
            pltpu.sync_copy(buf_v.at[1], s_sh.at[cidx_v.at[j0 + 1]],
                            add=True)
            return carry2

        lax.fori_loop(0, KI // 2, step, 0)
        return carry

    lax.fori_loop(0, CPW // KI, group, 0)
    plsc.subcore_barrier()
    pltpu.sync_copy(s_sh.at[pl.ds(s * RPS, RPS)],
                    out_hbm.at[c, pl.ds(s * RPS, RPS)])


@functools.cache
def _sc_kernels():
    mesh = plsc.VectorSubcoreMesh(core_axis_name="c", subcore_axis_name="s",
                                  num_cores=NC, num_subcores=NS)
    deg = pl.kernel(
        _sc_degree_body,
        out_type=jax.ShapeDtypeStruct((NC, NPAD), jnp.float32),
        mesh=mesh,
        scratch_types=[
            pltpu.VMEM((CPW, CHUNK), jnp.int32),
            pltpu.VMEM((CHUNK,), jnp.float32),
            pltpu.VMEM((RPS,), jnp.float32),
            pltpu.VMEM_SHARED((NPAD,), jnp.float32),
        ],
    )
    scatter = pl.kernel(
        _sc_scatter_body,
        out_type=jax.ShapeDtypeStruct((NC, NPAD, D), jnp.float32),
        mesh=mesh,
        scratch_types=[
            pltpu.VMEM((KI, CHUNK), jnp.int32),
            pltpu.VMEM((KI, CHUNK), jnp.int32),
            pltpu.VMEM((2, CHUNK, D), jnp.float32),
            pltpu.VMEM_SHARED((NPAD, D), jnp.float32),
            pltpu.SemaphoreType.DMA((2,)),
        ],
    )
    return deg, scatter


def _dis(degp_ref):
    return lax.rsqrt(degp_ref[0] + degp_ref[1] + 1.0)


def _mm1_body(x_ref, w_ref, degp_ref, out_ref):
    h = jnp.dot(x_ref[...], w_ref[...], preferred_element_type=jnp.float32)
    out_ref[...] = h * _dis(degp_ref)


def _pool_block(batch_ref, xl):
    bb = batch_ref[0]
    gids = lax.broadcasted_iota(jnp.int32, (G, CHUNK), 0)
    mask = (gids == bb).astype(jnp.float32)
    return jnp.dot(mask, xl, preferred_element_type=jnp.float32)


def _layer_body(s_ref, hp_ref, degp_ref, b_ref, w_ref, batch_ref,
                hpn_ref, p_ref):
    j = pl.program_id(0)
    dis = _dis(degp_ref)
    xl = jnp.maximum((s_ref[0] + s_ref[1] + hp_ref[...]) * dis + b_ref[...],
                     0.0)
    hpn_ref[...] = jnp.dot(xl, w_ref[...],
                           preferred_element_type=jnp.float32) * dis

    @pl.when(j == 0)
    def _():
        p_ref[...] = jnp.zeros_like(p_ref)

    p_ref[...] += _pool_block(batch_ref, xl)


def _final_body(s_ref, hp_ref, degp_ref, b_ref, batch_ref, p_ref):
    j = pl.program_id(0)
    dis = _dis(degp_ref)
    xl = jnp.maximum((s_ref[0] + s_ref[1] + hp_ref[...]) * dis + b_ref[...],
                     0.0)

    @pl.when(j == 0)
    def _():
        p_ref[...] = jnp.zeros_like(p_ref)

    p_ref[...] += _pool_block(batch_ref, xl)


_x_spec = pl.BlockSpec((CHUNK, D), lambda j: (j, 0))
_w_spec = pl.BlockSpec((D, D), lambda j: (0, 0))
_deg_spec = pl.BlockSpec((NC, CHUNK, 1), lambda j: (0, j, 0))
_s_spec = pl.BlockSpec((NC, CHUNK, D), lambda j: (0, j, 0))
_b_spec = pl.BlockSpec((1, D), lambda j: (0, 0))
_batch_spec = pl.BlockSpec((1, 1, CHUNK), lambda j: (j, 0, 0))
_p_spec = pl.BlockSpec((G, D), lambda j: (0, 0))

_mm1 = pl.pallas_call(
    _mm1_body,
    grid=(NB,),
    in_specs=[_x_spec, _w_spec, _deg_spec],
    out_specs=_x_spec,
    out_shape=jax.ShapeDtypeStruct((NPAD, D), jnp.float32),
)

_layer = pl.pallas_call(
    _layer_body,
    grid=(NB,),
    in_specs=[_s_spec, _x_spec, _deg_spec, _b_spec, _w_spec, _batch_spec],
    out_specs=[_x_spec, _p_spec],
    out_shape=[jax.ShapeDtypeStruct((NPAD, D), jnp.float32),
               jax.ShapeDtypeStruct((G, D), jnp.float32)],
)

_final = pl.pallas_call(
    _final_body,
    grid=(NB,),
    in_specs=[_s_spec, _x_spec, _deg_spec, _b_spec, _batch_spec],
    out_specs=_p_spec,
    out_shape=jax.ShapeDtypeStruct((G, D), jnp.float32),
)


def kernel(x, edge_index, batch, W1, b1, W2, b2, W3, b3):
    row = edge_index[0].astype(jnp.int32)
    col = edge_index[1].astype(jnp.int32)
    pad = jnp.arange(EPAD - E, dtype=jnp.int32) % (NPAD - N) + N
    rowp = jnp.concatenate([row, pad]).reshape(NW, CPW, CHUNK)
    colp = jnp.concatenate([col, pad]).reshape(NW, CPW, CHUNK)
    x_pad = jnp.pad(x, ((0, NPAD - N), (0, 0)))
    batch_pad = jnp.concatenate(
        [batch.astype(jnp.int32), jnp.full((NPAD - N,), -1, jnp.int32)]
    ).reshape(NB, 1, CHUNK)
    zeros = jnp.zeros((NPAD, D), jnp.float32)

    _sc_degree, _sc_scatter = _sc_kernels()
    degp = _sc_degree(colp).reshape(NC, NPAD, 1)
    hp1 = _mm1(x_pad, W1, degp)
    S1 = _sc_scatter(rowp, colp, hp1, zeros)
    hp2, p1 = _layer(S1, hp1, degp, b1.reshape(1, D), W2, batch_pad)
    S2 = _sc_scatter(rowp, colp, hp2, zeros)
    hp3, p2 = _layer(S2, hp2, degp, b2.reshape(1, D), W3, batch_pad)
    S3 = _sc_scatter(rowp, colp, hp3, zeros)
    p3 = _final(S3, hp3, degp, b3.reshape(1, D), batch_pad)
    return jnp.concatenate([p1, p2, p3], axis=1)
